# baseline (speedup 1.0000x reference)
import functools

import jax
import jax.numpy as jnp
from jax.experimental import pallas as pl
from jax.experimental.pallas import tpu as pltpu

N = 10000
F = 128
BM = 200


def _layer_body(adj_ref, xin_ref, w_ref, b_ref, out_ref, z_ref):
    @pl.when(pl.program_id(0) == 0)
    def _():
        z_ref[...] = jnp.dot(
            xin_ref[...], w_ref[...], preferred_element_type=jnp.float32
        )

    y = jnp.dot(adj_ref[...], z_ref[...], preferred_element_type=jnp.float32)
    y = y + b_ref[...]
    nrm = jnp.sqrt(jnp.sum(y * y, axis=1, keepdims=True))
    out_ref[...] = y / jnp.maximum(nrm, 1e-12)


@functools.partial(jax.jit, static_argnames=())
def _layer(adj, x_in, w, b):
    return pl.pallas_call(
        _layer_body,
        grid=(N // BM,),
        in_specs=[
            pl.BlockSpec((BM, N), lambda i: (i, 0)),
            pl.BlockSpec((N, F), lambda i: (0, 0)),
            pl.BlockSpec((F, F), lambda i: (0, 0)),
            pl.BlockSpec((1, F), lambda i: (0, 0)),
        ],
        out_specs=pl.BlockSpec((BM, F), lambda i: (i, 0)),
        out_shape=jax.ShapeDtypeStruct((N, F), jnp.float32),
        scratch_shapes=[pltpu.VMEM((N, F), jnp.float32)],
    )(adj, x_in, w, b)


def kernel(x, adj, W1, b1, W2, b2):
    h = _layer(adj, x, W1, b1.reshape(1, F))
    return _layer(adj, h, W2, b2.reshape(1, F))


# speedup vs baseline: 1.0069x; 1.0069x over previous
import functools

import jax
import jax.numpy as jnp
from jax.experimental import pallas as pl
from jax.experimental.pallas import tpu as pltpu

N = 10000
F = 128
BM = 400


def _layer_body(adj_ref, xin_ref, w_ref, b_ref, out_ref, z_ref):
    @pl.when(pl.program_id(0) == 0)
    def _():
        z_ref[...] = jnp.dot(
            xin_ref[...], w_ref[...], preferred_element_type=jnp.float32
        )

    y = jnp.dot(
        adj_ref[...].astype(jnp.bfloat16),
        z_ref[...].astype(jnp.bfloat16),
        preferred_element_type=jnp.float32,
    )
    y = y + b_ref[...]
    nrm = jnp.sqrt(jnp.sum(y * y, axis=1, keepdims=True))
    out_ref[...] = y / jnp.maximum(nrm, 1e-12)


@functools.partial(jax.jit, static_argnames=())
def _layer(adj, x_in, w, b):
    return pl.pallas_call(
        _layer_body,
        grid=(N // BM,),
        in_specs=[
            pl.BlockSpec((BM, N), lambda i: (i, 0)),
            pl.BlockSpec((N, F), lambda i: (0, 0)),
            pl.BlockSpec((F, F), lambda i: (0, 0)),
            pl.BlockSpec((1, F), lambda i: (0, 0)),
        ],
        out_specs=pl.BlockSpec((BM, F), lambda i: (i, 0)),
        out_shape=jax.ShapeDtypeStruct((N, F), jnp.float32),
        scratch_shapes=[pltpu.VMEM((N, F), jnp.float32)],
    )(adj, x_in, w, b)


def kernel(x, adj, W1, b1, W2, b2):
    h = _layer(adj, x, W1, b1.reshape(1, F))
    return _layer(adj, h, W2, b2.reshape(1, F))
